# baseline (speedup 1.0000x reference)
import jax
import jax.numpy as jnp
from jax import lax
from jax.experimental import pallas as pl
from jax.experimental.pallas import tpu as pltpu
from jax.experimental.pallas import tpu_sc as plsc

N = 10000
E = 320000
DM = 128
NP = 10240
EP = 327680
CHUNK = 128
NWORK = 32
CPT = EP // (NWORK * CHUNK)
RPT = NP // 16
CPT0 = 40
CPT1 = 120

def _mesh():
    return plsc.VectorSubcoreMesh(core_axis_name="c", subcore_axis_name="s",
                                  num_cores=2, num_subcores=16)


NBUF = 2


def _seg_body(src, gidx, sidx, zeros, out, idx_g, idx_s, rows,
              gs0, gs1, ss0, ss1, acc):
    gsem = (gs0, gs1)
    ssem = (ss0, ss1)
    c = lax.axis_index("c")
    s = lax.axis_index("s")
    row0 = s * RPT
    pltpu.sync_copy(zeros.at[pl.ds(row0, RPT)], acc.at[pl.ds(row0, RPT)])
    plsc.subcore_barrier()
    cpt = jnp.where(c == 0, CPT0, CPT1)
    base0 = jnp.where(c == 0, s * CPT0, 16 * CPT0 + s * CPT1) * CHUNK

    def start_gather(j, b):
        base = base0 + j * CHUNK
        pltpu.sync_copy(gidx.at[pl.ds(base, CHUNK)], idx_g.at[b])
        pltpu.sync_copy(sidx.at[pl.ds(base, CHUNK)], idx_s.at[b])
        pltpu.async_copy(src.at[idx_g.at[b]], rows.at[b], gsem[b])

    def wait_gather(b):
        pltpu.make_async_copy(src.at[pl.ds(0, CHUNK)], rows.at[b],
                              gsem[b]).wait()

    def start_scatter(b):
        pltpu.async_copy(rows.at[b], acc.at[idx_s.at[b]], ssem[b], add=True)

    def wait_scatter(b):
        pltpu.make_async_copy(rows.at[b], acc.at[pl.ds(0, CHUNK)],
                              ssem[b]).wait()

    HEAD = NBUF - 1
    start_gather(0, 0)
    for j in range(HEAD):
        start_gather(j + 1, (j + 1) % NBUF)
        wait_gather(j % NBUF)
        start_scatter(j % NBUF)

    def body(jj, carry):
        for t in range(NBUF):
            j = HEAD + jj * NBUF + t
            b = (HEAD + t) % NBUF
            bn = t % NBUF
            wait_scatter(bn)
            start_gather(j + 1, bn)
            wait_gather(b)
            start_scatter(b)
        return carry

    lax.fori_loop(0, (cpt - NBUF) // NBUF, body, 0)
    wait_gather(1)
    start_scatter(1)
    for b in range(NBUF):
        wait_scatter(b)
    plsc.subcore_barrier()
    pltpu.sync_copy(acc.at[pl.ds(row0, RPT)],
                    out.at[pl.ds(c * NP + row0, RPT)])


def _seg_rows(src, gidx, sidx, zeros):
    return pl.kernel(
        _seg_body,
        out_type=jax.ShapeDtypeStruct((2 * NP, DM), jnp.float32),
        mesh=_mesh(),
        scratch_types=[
            pltpu.VMEM((NBUF, CHUNK), jnp.int32),
            pltpu.VMEM((NBUF, CHUNK), jnp.int32),
            pltpu.VMEM((NBUF, CHUNK, DM), jnp.float32),
            pltpu.SemaphoreType.DMA,
            pltpu.SemaphoreType.DMA,
            pltpu.SemaphoreType.DMA,
            pltpu.SemaphoreType.DMA,
            pltpu.VMEM_SHARED((NP, DM), jnp.float32),
        ],
    )(src, gidx, sidx, zeros)


def _deg_body(nidx, eidx, we, zeros1, dout, bout,
              idx_n, idx_e, wbuf, ones, sem, dacc, bacc):
    c = lax.axis_index("c")
    s = lax.axis_index("s")
    row0 = s * RPT
    pltpu.sync_copy(zeros1.at[pl.ds(row0, RPT)], dacc.at[pl.ds(row0, RPT)])
    pltpu.sync_copy(zeros1.at[pl.ds(row0, RPT)], bacc.at[pl.ds(row0, RPT)])
    for i in range(CHUNK // 16):
        ones[pl.ds(i * 16, 16)] = jnp.ones((16,), jnp.float32)
    plsc.subcore_barrier()
    wid = s * 2 + c
    base0 = wid * CPT * CHUNK

    def body(j, carry):
        base = base0 + j * CHUNK
        pltpu.sync_copy(nidx.at[pl.ds(base, CHUNK)], idx_n)
        pltpu.sync_copy(eidx.at[pl.ds(base, CHUNK)], idx_e)
        pltpu.async_copy(we.at[idx_e], wbuf, sem).wait()
        pltpu.sync_copy(wbuf, dacc.at[idx_n], add=True)
        pltpu.sync_copy(ones, bacc.at[idx_e], add=True)
        return carry

    lax.fori_loop(0, CPT, body, 0)
    plsc.subcore_barrier()
    pltpu.sync_copy(dacc.at[pl.ds(row0, RPT)],
                    dout.at[pl.ds(c * NP + row0, RPT)])
    pltpu.sync_copy(bacc.at[pl.ds(row0, RPT)],
                    bout.at[pl.ds(c * NP + row0, RPT)])


def _degrees(nidx, eidx, we, zeros1):
    return pl.kernel(
        _deg_body,
        out_type=(jax.ShapeDtypeStruct((2 * NP,), jnp.float32),
                  jax.ShapeDtypeStruct((2 * NP,), jnp.float32)),
        mesh=_mesh(),
        scratch_types=[
            pltpu.VMEM((CHUNK,), jnp.int32),
            pltpu.VMEM((CHUNK,), jnp.int32),
            pltpu.VMEM((CHUNK,), jnp.float32),
            pltpu.VMEM((CHUNK,), jnp.float32),
            pltpu.SemaphoreType.DMA,
            pltpu.VMEM_SHARED((NP,), jnp.float32),
            pltpu.VMEM_SHARED((NP,), jnp.float32),
        ],
    )(nidx, eidx, we, zeros1)


BLKR = 512
GRID = NP // BLKR


def _mm_body(x_ref, w_ref, o_ref):
    o_ref[...] = jnp.dot(x_ref[...], w_ref[...],
                         preferred_element_type=jnp.float32)


def _matmul(x, w):
    return pl.pallas_call(
        _mm_body,
        grid=(GRID,),
        in_specs=[pl.BlockSpec((BLKR, DM), lambda i: (i, 0)),
                  pl.BlockSpec((DM, DM), lambda i: (0, 0))],
        out_specs=pl.BlockSpec((BLKR, DM), lambda i: (i, 0)),
        out_shape=jax.ShapeDtypeStruct((NP, DM), jnp.float32),
    )(x, w)


def _escale_body(p0, p1, b0, b1, we_ref, o_ref):
    bdeg = b0[...] + b1[...]
    safe = jnp.where(bdeg > 0, bdeg, 1.0)
    scale = jnp.where(bdeg > 0, we_ref[...] / safe, 0.0)
    o_ref[...] = (p0[...] + p1[...]) * scale


def _escale(p, bpart, we2):
    return pl.pallas_call(
        _escale_body,
        grid=(GRID,),
        in_specs=[pl.BlockSpec((BLKR, DM), lambda i: (i, 0)),
                  pl.BlockSpec((BLKR, DM), lambda i: (i + GRID, 0)),
                  pl.BlockSpec((BLKR, 1), lambda i: (i, 0)),
                  pl.BlockSpec((BLKR, 1), lambda i: (i + GRID, 0)),
                  pl.BlockSpec((BLKR, 1), lambda i: (i, 0))],
        out_specs=pl.BlockSpec((BLKR, DM), lambda i: (i, 0)),
        out_shape=jax.ShapeDtypeStruct((NP, DM), jnp.float32),
    )(p, p, bpart, bpart, we2)


def _mid_body(p0, p1, d0, d1, b1v, g_ref, be_ref, w2_ref, o_ref):
    deg = d0[...] + d1[...]
    safe = jnp.where(deg > 0, deg, 1.0)
    dinv = jnp.where(deg > 0, 1.0 / safe, 0.0)
    h = (p0[...] + p1[...]) * dinv + b1v[...]
    h = jnp.maximum(h, 0.0)
    mu = jnp.mean(h, axis=1, keepdims=True)
    var = jnp.mean((h - mu) * (h - mu), axis=1, keepdims=True)
    h = (h - mu) / jnp.sqrt(var + 1e-5) * g_ref[...] + be_ref[...]
    o_ref[...] = jnp.dot(h, w2_ref[...], preferred_element_type=jnp.float32)


def _mid(p, dpart, b1, gamma, beta, W2):
    return pl.pallas_call(
        _mid_body,
        grid=(GRID,),
        in_specs=[pl.BlockSpec((BLKR, DM), lambda i: (i, 0)),
                  pl.BlockSpec((BLKR, DM), lambda i: (i + GRID, 0)),
                  pl.BlockSpec((BLKR, 1), lambda i: (i, 0)),
                  pl.BlockSpec((BLKR, 1), lambda i: (i + GRID, 0)),
                  pl.BlockSpec((1, DM), lambda i: (0, 0)),
                  pl.BlockSpec((1, DM), lambda i: (0, 0)),
                  pl.BlockSpec((1, DM), lambda i: (0, 0)),
                  pl.BlockSpec((DM, DM), lambda i: (0, 0))],
        out_specs=pl.BlockSpec((BLKR, DM), lambda i: (i, 0)),
        out_shape=jax.ShapeDtypeStruct((NP, DM), jnp.float32),
    )(p, p, dpart, dpart, b1.reshape(1, DM), gamma.reshape(1, DM),
      beta.reshape(1, DM), W2)


def _final_body(p0, p1, d0, d1, b2v, x_ref, o_ref):
    deg = d0[...] + d1[...]
    safe = jnp.where(deg > 0, deg, 1.0)
    dinv = jnp.where(deg > 0, 1.0 / safe, 0.0)
    o_ref[...] = (p0[...] + p1[...]) * dinv + b2v[...] + x_ref[...]


def _final(p, dpart, b2, x):
    return pl.pallas_call(
        _final_body,
        grid=(GRID,),
        in_specs=[pl.BlockSpec((BLKR, DM), lambda i: (i, 0)),
                  pl.BlockSpec((BLKR, DM), lambda i: (i + GRID, 0)),
                  pl.BlockSpec((BLKR, 1), lambda i: (i, 0)),
                  pl.BlockSpec((BLKR, 1), lambda i: (i + GRID, 0)),
                  pl.BlockSpec((1, DM), lambda i: (0, 0)),
                  pl.BlockSpec((BLKR, DM), lambda i: (i, 0))],
        out_specs=pl.BlockSpec((BLKR, DM), lambda i: (i, 0)),
        out_shape=jax.ShapeDtypeStruct((NP, DM), jnp.float32),
    )(p, p, dpart, dpart, b2.reshape(1, DM), x)


def kernel(x, edge_index, edge_attr, W1, b1, W2, b2, gamma, beta):
    xpad = jnp.pad(x, ((0, NP - N), (0, 0)))
    pad = jnp.full((EP - E,), N, jnp.int32)
    nidx = jnp.concatenate([edge_index[0], pad])
    eidx = jnp.concatenate([edge_index[1], pad])
    wepad = jnp.pad(edge_attr, (0, NP - N))
    zeros2 = jnp.zeros((NP, DM), jnp.float32)
    zeros1 = jnp.zeros((NP,), jnp.float32)

    dpart, bpart = _degrees(nidx, eidx, wepad, zeros1)
    dpart = dpart.reshape(2 * NP, 1)
    bpart = bpart.reshape(2 * NP, 1)
    we2 = wepad.reshape(NP, 1)

    xt1 = _matmul(xpad, W1)
    pA1 = _seg_rows(xt1, nidx, eidx, zeros2)
    e1 = _escale(pA1, bpart, we2)
    pB1 = _seg_rows(e1, eidx, nidx, zeros2)
    xt2 = _mid(pB1, dpart, b1, gamma, beta, W2)

    pA2 = _seg_rows(xt2, nidx, eidx, zeros2)
    e2 = _escale(pA2, bpart, we2)
    pB2 = _seg_rows(e2, eidx, nidx, zeros2)
    out = _final(pB2, dpart, b2, xpad)
    return out[:N]


# speedup vs baseline: 1.1747x; 1.1747x over previous
import jax
import jax.numpy as jnp
from jax import lax
from jax.experimental import pallas as pl
from jax.experimental.pallas import tpu as pltpu
from jax.experimental.pallas import tpu_sc as plsc

N = 10000
E = 320000
DM = 128
NP = 10240
EP = 327680
CHUNK = 128
NWORK = 32
CPT = EP // (NWORK * CHUNK)
RPT = NP // 16
CPT0 = 120
CPT1 = 40

def _mesh():
    return plsc.VectorSubcoreMesh(core_axis_name="c", subcore_axis_name="s",
                                  num_cores=2, num_subcores=16)


NBUF = 2


def _seg_body(src, gidx, sidx, zeros, out, idx_g, idx_s, rows,
              gs0, gs1, ss0, ss1, acc):
    gsem = (gs0, gs1)
    ssem = (ss0, ss1)
    c = lax.axis_index("c")
    s = lax.axis_index("s")
    row0 = s * RPT
    pltpu.sync_copy(zeros.at[pl.ds(row0, RPT)], acc.at[pl.ds(row0, RPT)])
    plsc.subcore_barrier()
    cpt = jnp.where(c == 0, CPT0, CPT1)
    base0 = jnp.where(c == 0, s * CPT0, 16 * CPT0 + s * CPT1) * CHUNK

    def start_gather(j, b):
        base = base0 + j * CHUNK
        pltpu.sync_copy(gidx.at[pl.ds(base, CHUNK)], idx_g.at[b])
        pltpu.sync_copy(sidx.at[pl.ds(base, CHUNK)], idx_s.at[b])
        pltpu.async_copy(src.at[idx_g.at[b]], rows.at[b], gsem[b])

    def wait_gather(b):
        pltpu.make_async_copy(src.at[pl.ds(0, CHUNK)], rows.at[b],
                              gsem[b]).wait()

    def start_scatter(b):
        pltpu.async_copy(rows.at[b], acc.at[idx_s.at[b]], ssem[b], add=True)

    def wait_scatter(b):
        pltpu.make_async_copy(rows.at[b], acc.at[pl.ds(0, CHUNK)],
                              ssem[b]).wait()

    HEAD = NBUF - 1
    start_gather(0, 0)
    for j in range(HEAD):
        start_gather(j + 1, (j + 1) % NBUF)
        wait_gather(j % NBUF)
        start_scatter(j % NBUF)

    def body(jj, carry):
        for t in range(NBUF):
            j = HEAD + jj * NBUF + t
            b = (HEAD + t) % NBUF
            bn = t % NBUF
            wait_scatter(bn)
            start_gather(j + 1, bn)
            wait_gather(b)
            start_scatter(b)
        return carry

    lax.fori_loop(0, (cpt - NBUF) // NBUF, body, 0)
    wait_gather(1)
    start_scatter(1)
    for b in range(NBUF):
        wait_scatter(b)
    plsc.subcore_barrier()
    pltpu.sync_copy(acc.at[pl.ds(row0, RPT)],
                    out.at[pl.ds(c * NP + row0, RPT)])


def _seg_rows(src, gidx, sidx, zeros):
    return pl.kernel(
        _seg_body,
        out_type=jax.ShapeDtypeStruct((2 * NP, DM), jnp.float32),
        mesh=_mesh(),
        scratch_types=[
            pltpu.VMEM((NBUF, CHUNK), jnp.int32),
            pltpu.VMEM((NBUF, CHUNK), jnp.int32),
            pltpu.VMEM((NBUF, CHUNK, DM), jnp.float32),
            pltpu.SemaphoreType.DMA,
            pltpu.SemaphoreType.DMA,
            pltpu.SemaphoreType.DMA,
            pltpu.SemaphoreType.DMA,
            pltpu.VMEM_SHARED((NP, DM), jnp.float32),
        ],
    )(src, gidx, sidx, zeros)


def _deg_body(nidx, eidx, we, zeros1, dout, bout,
              idx_n, idx_e, wbuf, ones, sem, dacc, bacc):
    c = lax.axis_index("c")
    s = lax.axis_index("s")
    row0 = s * RPT
    pltpu.sync_copy(zeros1.at[pl.ds(row0, RPT)], dacc.at[pl.ds(row0, RPT)])
    pltpu.sync_copy(zeros1.at[pl.ds(row0, RPT)], bacc.at[pl.ds(row0, RPT)])
    for i in range(CHUNK // 16):
        ones[pl.ds(i * 16, 16)] = jnp.ones((16,), jnp.float32)
    plsc.subcore_barrier()
    wid = s * 2 + c
    base0 = wid * CPT * CHUNK

    def body(j, carry):
        base = base0 + j * CHUNK
        pltpu.sync_copy(nidx.at[pl.ds(base, CHUNK)], idx_n)
        pltpu.sync_copy(eidx.at[pl.ds(base, CHUNK)], idx_e)
        pltpu.async_copy(we.at[idx_e], wbuf, sem).wait()
        pltpu.sync_copy(wbuf, dacc.at[idx_n], add=True)
        pltpu.sync_copy(ones, bacc.at[idx_e], add=True)
        return carry

    lax.fori_loop(0, CPT, body, 0)
    plsc.subcore_barrier()
    pltpu.sync_copy(dacc.at[pl.ds(row0, RPT)],
                    dout.at[pl.ds(c * NP + row0, RPT)])
    pltpu.sync_copy(bacc.at[pl.ds(row0, RPT)],
                    bout.at[pl.ds(c * NP + row0, RPT)])


def _degrees(nidx, eidx, we, zeros1):
    return pl.kernel(
        _deg_body,
        out_type=(jax.ShapeDtypeStruct((2 * NP,), jnp.float32),
                  jax.ShapeDtypeStruct((2 * NP,), jnp.float32)),
        mesh=_mesh(),
        scratch_types=[
            pltpu.VMEM((CHUNK,), jnp.int32),
            pltpu.VMEM((CHUNK,), jnp.int32),
            pltpu.VMEM((CHUNK,), jnp.float32),
            pltpu.VMEM((CHUNK,), jnp.float32),
            pltpu.SemaphoreType.DMA,
            pltpu.VMEM_SHARED((NP,), jnp.float32),
            pltpu.VMEM_SHARED((NP,), jnp.float32),
        ],
    )(nidx, eidx, we, zeros1)


BLKR = 512
GRID = NP // BLKR


def _mm_body(x_ref, w_ref, o_ref):
    o_ref[...] = jnp.dot(x_ref[...], w_ref[...],
                         preferred_element_type=jnp.float32)


def _matmul(x, w):
    return pl.pallas_call(
        _mm_body,
        grid=(GRID,),
        in_specs=[pl.BlockSpec((BLKR, DM), lambda i: (i, 0)),
                  pl.BlockSpec((DM, DM), lambda i: (0, 0))],
        out_specs=pl.BlockSpec((BLKR, DM), lambda i: (i, 0)),
        out_shape=jax.ShapeDtypeStruct((NP, DM), jnp.float32),
    )(x, w)


def _escale_body(p0, p1, b0, b1, we_ref, o_ref):
    bdeg = b0[...] + b1[...]
    safe = jnp.where(bdeg > 0, bdeg, 1.0)
    scale = jnp.where(bdeg > 0, we_ref[...] / safe, 0.0)
    o_ref[...] = (p0[...] + p1[...]) * scale


def _escale(p, bpart, we2):
    return pl.pallas_call(
        _escale_body,
        grid=(GRID,),
        in_specs=[pl.BlockSpec((BLKR, DM), lambda i: (i, 0)),
                  pl.BlockSpec((BLKR, DM), lambda i: (i + GRID, 0)),
                  pl.BlockSpec((BLKR, 1), lambda i: (i, 0)),
                  pl.BlockSpec((BLKR, 1), lambda i: (i + GRID, 0)),
                  pl.BlockSpec((BLKR, 1), lambda i: (i, 0))],
        out_specs=pl.BlockSpec((BLKR, DM), lambda i: (i, 0)),
        out_shape=jax.ShapeDtypeStruct((NP, DM), jnp.float32),
    )(p, p, bpart, bpart, we2)


def _mid_body(p0, p1, d0, d1, b1v, g_ref, be_ref, w2_ref, o_ref):
    deg = d0[...] + d1[...]
    safe = jnp.where(deg > 0, deg, 1.0)
    dinv = jnp.where(deg > 0, 1.0 / safe, 0.0)
    h = (p0[...] + p1[...]) * dinv + b1v[...]
    h = jnp.maximum(h, 0.0)
    mu = jnp.mean(h, axis=1, keepdims=True)
    var = jnp.mean((h - mu) * (h - mu), axis=1, keepdims=True)
    h = (h - mu) / jnp.sqrt(var + 1e-5) * g_ref[...] + be_ref[...]
    o_ref[...] = jnp.dot(h, w2_ref[...], preferred_element_type=jnp.float32)


def _mid(p, dpart, b1, gamma, beta, W2):
    return pl.pallas_call(
        _mid_body,
        grid=(GRID,),
        in_specs=[pl.BlockSpec((BLKR, DM), lambda i: (i, 0)),
                  pl.BlockSpec((BLKR, DM), lambda i: (i + GRID, 0)),
                  pl.BlockSpec((BLKR, 1), lambda i: (i, 0)),
                  pl.BlockSpec((BLKR, 1), lambda i: (i + GRID, 0)),
                  pl.BlockSpec((1, DM), lambda i: (0, 0)),
                  pl.BlockSpec((1, DM), lambda i: (0, 0)),
                  pl.BlockSpec((1, DM), lambda i: (0, 0)),
                  pl.BlockSpec((DM, DM), lambda i: (0, 0))],
        out_specs=pl.BlockSpec((BLKR, DM), lambda i: (i, 0)),
        out_shape=jax.ShapeDtypeStruct((NP, DM), jnp.float32),
    )(p, p, dpart, dpart, b1.reshape(1, DM), gamma.reshape(1, DM),
      beta.reshape(1, DM), W2)


def _final_body(p0, p1, d0, d1, b2v, x_ref, o_ref):
    deg = d0[...] + d1[...]
    safe = jnp.where(deg > 0, deg, 1.0)
    dinv = jnp.where(deg > 0, 1.0 / safe, 0.0)
    o_ref[...] = (p0[...] + p1[...]) * dinv + b2v[...] + x_ref[...]


def _final(p, dpart, b2, x):
    return pl.pallas_call(
        _final_body,
        grid=(GRID,),
        in_specs=[pl.BlockSpec((BLKR, DM), lambda i: (i, 0)),
                  pl.BlockSpec((BLKR, DM), lambda i: (i + GRID, 0)),
                  pl.BlockSpec((BLKR, 1), lambda i: (i, 0)),
                  pl.BlockSpec((BLKR, 1), lambda i: (i + GRID, 0)),
                  pl.BlockSpec((1, DM), lambda i: (0, 0)),
                  pl.BlockSpec((BLKR, DM), lambda i: (i, 0))],
        out_specs=pl.BlockSpec((BLKR, DM), lambda i: (i, 0)),
        out_shape=jax.ShapeDtypeStruct((NP, DM), jnp.float32),
    )(p, p, dpart, dpart, b2.reshape(1, DM), x)


def kernel(x, edge_index, edge_attr, W1, b1, W2, b2, gamma, beta):
    xpad = jnp.pad(x, ((0, NP - N), (0, 0)))
    pad = jnp.full((EP - E,), N, jnp.int32)
    nidx = jnp.concatenate([edge_index[0], pad])
    eidx = jnp.concatenate([edge_index[1], pad])
    wepad = jnp.pad(edge_attr, (0, NP - N))
    zeros2 = jnp.zeros((NP, DM), jnp.float32)
    zeros1 = jnp.zeros((NP,), jnp.float32)

    dpart, bpart = _degrees(nidx, eidx, wepad, zeros1)
    dpart = dpart.reshape(2 * NP, 1)
    bpart = bpart.reshape(2 * NP, 1)
    we2 = wepad.reshape(NP, 1)

    xt1 = _matmul(xpad, W1)
    pA1 = _seg_rows(xt1, nidx, eidx, zeros2)
    e1 = _escale(pA1, bpart, we2)
    pB1 = _seg_rows(e1, eidx, nidx, zeros2)
    xt2 = _mid(pB1, dpart, b1, gamma, beta, W2)

    pA2 = _seg_rows(xt2, nidx, eidx, zeros2)
    e2 = _escale(pA2, bpart, we2)
    pB2 = _seg_rows(e2, eidx, nidx, zeros2)
    out = _final(pB2, dpart, b2, xpad)
    return out[:N]
